# baseline (speedup 1.0000x reference)
import functools

import jax
import jax.numpy as jnp
from jax import lax
from jax.experimental import pallas as pl
from jax.experimental.pallas import tpu as pltpu
from jax.experimental.pallas import tpu_sc as plsc

N_NODES = 10000
N_PAD = 10240
ROW_BLK = 512
HID = 512
IN_DIM = 256
N_CLASSES = 128
QW = 128
NQ = HID // QW

N_EDGES = 160000
CHUNK = 128
N_CHUNKS = 1280
E_CAP = N_CHUNKS * CHUNK
NS = 16
CPT = N_CHUNKS // NS

NH = N_PAD // 2
SCR = 1024
ACC_ROWS = NH + SCR
ZR = ACC_ROWS // NS
FR = NH // NS

_sc_mesh = plsc.VectorSubcoreMesh(core_axis_name="c", subcore_axis_name="s")


def _make_deg_kernel():
    @functools.partial(
        pl.kernel,
        mesh=_sc_mesh,
        out_type=jax.ShapeDtypeStruct((2, N_PAD, QW), jnp.float32),
        scratch_types=[
            pltpu.VMEM((CPT, CHUNK), jnp.int32),
            pltpu.VMEM((CPT, CHUNK), jnp.int32),
            pltpu.VMEM((CHUNK, QW), jnp.float32),
            pltpu.VMEM_SHARED((ACC_ROWS, QW), jnp.float32),
        ],
    )
    def deg_kernel(idx_hbm, ones_hbm, zeros_hbm, out_hbm,
                   idx_v, iloc_v, ones_v, acc_sh):
        cid = lax.axis_index("c")
        sid = lax.axis_index("s")
        pltpu.sync_copy(ones_hbm, ones_v)
        pltpu.sync_copy(idx_hbm.at[cid].at[pl.ds(sid * CPT, CPT)], idx_v)
        for b in range(2):
            @pl.loop(0, CPT)
            def _(r):
                @pl.loop(0, CHUNK // 16)
                def _(c):
                    v = idx_v[r, pl.ds(c * 16, 16)]
                    loc = v - b * NH
                    ok = (loc >= 0) & (loc < NH)
                    scr = NH + jnp.bitwise_and(v, SCR - 1)
                    iloc_v[r, pl.ds(c * 16, 16)] = jnp.where(ok, loc, scr)

            pltpu.sync_copy(zeros_hbm, acc_sh.at[pl.ds(sid * ZR, ZR)])
            plsc.subcore_barrier()

            @pl.loop(0, CPT)
            def _(i):
                pltpu.sync_copy(ones_v, acc_sh.at[iloc_v.at[i]], add=True)

            plsc.subcore_barrier()
            pltpu.sync_copy(
                acc_sh.at[pl.ds(sid * FR, FR)],
                out_hbm.at[cid].at[pl.ds(b * NH + sid * FR, FR)])
            plsc.subcore_barrier()

    return deg_kernel


def _make_agg_kernel():
    HCPT = CPT // 2

    @functools.partial(
        pl.kernel,
        mesh=_sc_mesh,
        out_type=jax.ShapeDtypeStruct((NQ, N_PAD, QW), jnp.float32),
        scratch_types=[
            pltpu.VMEM((CPT, CHUNK), jnp.int32),
            pltpu.VMEM((CPT, CHUNK), jnp.int32),
            pltpu.VMEM((CPT, CHUNK), jnp.int32),
            pltpu.VMEM((CHUNK, QW), jnp.float32),
            pltpu.VMEM((CHUNK, QW), jnp.float32),
            pltpu.VMEM_SHARED((ACC_ROWS, QW), jnp.float32),
            pltpu.SemaphoreType.DMA,
            pltpu.SemaphoreType.DMA,
            pltpu.SemaphoreType.DMA,
        ],
    )
    def agg_kernel(tab_hbm, src_hbm, dst_hbm, zeros_hbm, out_hbm,
                   src_v, dst_v, dloc_v, rows0, rows1, acc_sh,
                   sem0, sem1, ssem):
        cid = lax.axis_index("c")
        sid = lax.axis_index("s")
        pltpu.sync_copy(src_hbm.at[pl.ds(sid * CPT, CPT)], src_v)
        pltpu.sync_copy(dst_hbm.at[pl.ds(sid * CPT, CPT)], dst_v)
        for b in range(2):
            @pl.loop(0, CPT)
            def _(r):
                @pl.loop(0, CHUNK // 16)
                def _(c):
                    v = dst_v[r, pl.ds(c * 16, 16)]
                    loc = v - b * NH
                    ok = (loc >= 0) & (loc < NH)
                    scr = NH + jnp.bitwise_and(v, SCR - 1)
                    dloc_v[r, pl.ds(c * 16, 16)] = jnp.where(ok, loc, scr)

            for q in range(NQ // 2):
                qg = cid * (NQ // 2) + q
                tab_q = tab_hbm.at[qg]
                pltpu.sync_copy(zeros_hbm, acc_sh.at[pl.ds(sid * ZR, ZR)])
                plsc.subcore_barrier()
                pltpu.async_copy(tab_q.at[src_v.at[0]], rows0, sem0)

                @pl.loop(0, HCPT)
                def _(j):
                    i0 = 2 * j
                    pltpu.async_copy(tab_q.at[src_v.at[i0 + 1]], rows1, sem1)
                    pltpu.make_async_copy(
                        tab_q.at[src_v.at[i0]], rows0, sem0).wait()
                    pltpu.async_copy(rows0, acc_sh.at[dloc_v.at[i0]], ssem,
                                     add=True)
                    pltpu.make_async_copy(
                        tab_q.at[src_v.at[i0 + 1]], rows1, sem1).wait()
                    pltpu.async_copy(rows1, acc_sh.at[dloc_v.at[i0 + 1]],
                                     ssem, add=True)
                    pltpu.make_async_copy(
                        tab_q.at[pl.ds(0, CHUNK)], rows0, ssem).wait()
                    pltpu.make_async_copy(
                        tab_q.at[pl.ds(0, CHUNK)], rows1, ssem).wait()

                    @pl.when(j < HCPT - 1)
                    def _():
                        pltpu.async_copy(
                            tab_q.at[src_v.at[i0 + 2]], rows0, sem0)

                plsc.subcore_barrier()
                pltpu.sync_copy(
                    acc_sh.at[pl.ds(sid * FR, FR)],
                    out_hbm.at[qg].at[pl.ds(b * NH + sid * FR, FR)])
                plsc.subcore_barrier()

    return agg_kernel


def _prep_kernel(feat_ref, dego_ref, degi_ref, ft_ref, a_ref, b_ref):
    a = lax.rsqrt(jnp.maximum(dego_ref[0][:, :1], 1.0))
    b = lax.rsqrt(jnp.maximum(degi_ref[0][:, :1], 1.0))
    a_ref[...] = a
    b_ref[...] = b
    ft = feat_ref[...] * a
    for q in range(NQ):
        if q < IN_DIM // QW:
            ft_ref[q] = ft[:, q * QW:(q + 1) * QW]
        else:
            ft_ref[q] = jnp.zeros((ft.shape[0], QW), jnp.float32)


def _prep(features_p, deg16):
    return pl.pallas_call(
        _prep_kernel,
        grid=(N_PAD // ROW_BLK,),
        in_specs=[
            pl.BlockSpec((ROW_BLK, IN_DIM), lambda i: (i, 0)),
            pl.BlockSpec((1, ROW_BLK, QW), lambda i: (0, i, 0)),
            pl.BlockSpec((1, ROW_BLK, QW), lambda i: (1, i, 0)),
        ],
        out_specs=[
            pl.BlockSpec((NQ, ROW_BLK, QW), lambda i: (0, i, 0)),
            pl.BlockSpec((ROW_BLK, 1), lambda i: (i, 0)),
            pl.BlockSpec((ROW_BLK, 1), lambda i: (i, 0)),
        ],
        out_shape=[
            jax.ShapeDtypeStruct((NQ, N_PAD, QW), jnp.float32),
            jax.ShapeDtypeStruct((N_PAD, 1), jnp.float32),
            jax.ShapeDtypeStruct((N_PAD, 1), jnp.float32),
        ],
    )(features_p, deg16, deg16)


def _layer_kernel(emit_ht, s_ref, a_ref, b_ref, w_ref, bias_ref, *out):
    x = jnp.concatenate([s_ref[q] for q in range(NQ)], axis=-1)
    x = x * b_ref[...]
    h = jnp.maximum(
        jnp.dot(x, w_ref[...], preferred_element_type=jnp.float32)
        + bias_ref[...], 0.0)
    out[0][...] = h
    if emit_ht:
        ht = h * a_ref[...]
        for q in range(NQ):
            out[1][q] = ht[:, q * QW:(q + 1) * QW]


def _layer(s, a_col, b_col, W, bias, emit_ht):
    out_shape = [jax.ShapeDtypeStruct((N_PAD, HID), jnp.float32)]
    out_specs = [pl.BlockSpec((ROW_BLK, HID), lambda i: (i, 0))]
    if emit_ht:
        out_shape.append(
            jax.ShapeDtypeStruct((NQ, N_PAD, QW), jnp.float32))
        out_specs.append(
            pl.BlockSpec((NQ, ROW_BLK, QW), lambda i: (0, i, 0)))
    return pl.pallas_call(
        functools.partial(_layer_kernel, emit_ht),
        grid=(N_PAD // ROW_BLK,),
        in_specs=[
            pl.BlockSpec((NQ, ROW_BLK, QW), lambda i: (0, i, 0)),
            pl.BlockSpec((ROW_BLK, 1), lambda i: (i, 0)),
            pl.BlockSpec((ROW_BLK, 1), lambda i: (i, 0)),
            pl.BlockSpec((HID, HID), lambda i: (0, 0)),
            pl.BlockSpec((1, HID), lambda i: (0, 0)),
        ],
        out_specs=out_specs,
        out_shape=out_shape,
    )(s, a_col, b_col, W, bias.reshape(1, HID))


def _pool_kernel(h1_ref, h2_ref, h3_ref, h4_ref, wa_ref, ba_ref, wo_ref,
                 bo_ref, out_ref):
    hs = [h1_ref[...], h2_ref[...], h3_ref[...], h4_ref[...]]
    wa = wa_ref[...]
    scores = [jnp.sum(h * wa, axis=1, keepdims=True) + ba_ref[0, 0] for h in hs]
    m = functools.reduce(jnp.maximum, scores)
    es = [jnp.exp(s - m) for s in scores]
    denom = functools.reduce(jnp.add, es)
    pooled = functools.reduce(
        jnp.add, [h * (e / denom) for h, e in zip(hs, es)])
    out_ref[...] = jnp.dot(pooled, wo_ref[...],
                           preferred_element_type=jnp.float32) + bo_ref[...]


def _pool_out(h1, h2, h3, h4, Wa, ba, Wo, bo):
    return pl.pallas_call(
        _pool_kernel,
        grid=(N_PAD // ROW_BLK,),
        in_specs=[pl.BlockSpec((ROW_BLK, HID), lambda i: (i, 0))] * 4 + [
            pl.BlockSpec((1, HID), lambda i: (0, 0)),
            pl.BlockSpec((1, 1), lambda i: (0, 0), memory_space=pltpu.SMEM),
            pl.BlockSpec((HID, N_CLASSES), lambda i: (0, 0)),
            pl.BlockSpec((1, N_CLASSES), lambda i: (0, 0)),
        ],
        out_specs=pl.BlockSpec((ROW_BLK, N_CLASSES), lambda i: (i, 0)),
        out_shape=jax.ShapeDtypeStruct((N_PAD, N_CLASSES), jnp.float32),
    )(h1, h2, h3, h4, Wa.reshape(1, HID), ba.reshape(1, 1), Wo,
      bo.reshape(1, N_CLASSES))


def kernel(features, edge_index, W0, b0, W1, b1, W2, b2, W3, b3, Wa, ba, Wo, bo):
    src = edge_index[0].astype(jnp.int32)
    dst = edge_index[1].astype(jnp.int32)
    n_fill = E_CAP - N_EDGES
    ar = jnp.arange(n_fill, dtype=jnp.int32)
    src2d = jnp.concatenate(
        [src, N_NODES + (ar % (N_PAD - N_NODES))]).reshape(N_CHUNKS, CHUNK)
    dst2d = jnp.concatenate(
        [dst, 2 * N_PAD + (ar % SCR)]).reshape(N_CHUNKS, CHUNK)
    idx2 = jnp.stack([src2d, dst2d])

    zeros_acc = jnp.zeros((ZR, QW), jnp.float32)
    ones128 = jnp.ones((CHUNK, QW), jnp.float32)
    feats_p = jnp.pad(features, ((0, N_PAD - N_NODES), (0, 0)))
    W0p = jnp.pad(W0, ((0, HID - IN_DIM), (0, 0)))

    deg16 = _make_deg_kernel()(idx2, ones128, zeros_acc)

    ft, a_col, b_col = _prep(feats_p, deg16)

    agg = _make_agg_kernel()

    s0 = agg(ft, src2d, dst2d, zeros_acc)
    h1, ht1 = _layer(s0, a_col, b_col, W0p, b0, True)
    s1 = agg(ht1, src2d, dst2d, zeros_acc)
    h2, ht2 = _layer(s1, a_col, b_col, W1, b1, True)
    s2 = agg(ht2, src2d, dst2d, zeros_acc)
    h3, ht3 = _layer(s2, a_col, b_col, W2, b2, True)
    s3 = agg(ht3, src2d, dst2d, zeros_acc)
    (h4,) = _layer(s3, a_col, b_col, W3, b3, False)

    out = _pool_out(h1, h2, h3, h4, Wa, ba, Wo, bo)
    return out[:N_NODES]


# speedup vs baseline: 1.3458x; 1.3458x over previous
import functools

import jax
import jax.numpy as jnp
from jax import lax
from jax.experimental import pallas as pl
from jax.experimental.pallas import tpu as pltpu
from jax.experimental.pallas import tpu_sc as plsc

N_NODES = 10000
N_PAD = 10240
ROW_BLK = 512
HID = 512
IN_DIM = 256
N_CLASSES = 128
QW = 128
NQ = HID // QW

N_EDGES = 160000
CHUNK = 128
N_CHUNKS = 1280
E_CAP = N_CHUNKS * CHUNK
NS = 16
CPT = N_CHUNKS // NS

NH = N_PAD // 2
SCR = 1024
ACC_ROWS = NH + SCR
ZR = ACC_ROWS // NS
FR = NH // NS

_sc_mesh = plsc.VectorSubcoreMesh(core_axis_name="c", subcore_axis_name="s")


def _make_deg_kernel():
    @functools.partial(
        pl.kernel,
        mesh=_sc_mesh,
        out_type=jax.ShapeDtypeStruct((2, N_PAD, QW), jnp.float32),
        scratch_types=[
            pltpu.VMEM((CPT, CHUNK), jnp.int32),
            pltpu.VMEM((CPT, CHUNK), jnp.int32),
            pltpu.VMEM((CHUNK, QW), jnp.float32),
            pltpu.VMEM_SHARED((ACC_ROWS, QW), jnp.float32),
        ],
    )
    def deg_kernel(idx_hbm, ones_hbm, zeros_hbm, out_hbm,
                   idx_v, iloc_v, ones_v, acc_sh):
        cid = lax.axis_index("c")
        sid = lax.axis_index("s")
        pltpu.sync_copy(ones_hbm, ones_v)
        pltpu.sync_copy(idx_hbm.at[cid].at[pl.ds(sid * CPT, CPT)], idx_v)
        for b in range(2):
            @pl.loop(0, CPT)
            def _(r):
                @pl.loop(0, CHUNK // 16)
                def _(c):
                    v = idx_v[r, pl.ds(c * 16, 16)]
                    loc = v - b * NH
                    ok = (loc >= 0) & (loc < NH)
                    scr = NH + jnp.bitwise_and(v, SCR - 1)
                    iloc_v[r, pl.ds(c * 16, 16)] = jnp.where(ok, loc, scr)

            pltpu.sync_copy(zeros_hbm, acc_sh.at[pl.ds(sid * ZR, ZR)])
            plsc.subcore_barrier()

            @pl.loop(0, CPT)
            def _(i):
                pltpu.sync_copy(ones_v, acc_sh.at[iloc_v.at[i]], add=True)

            plsc.subcore_barrier()
            pltpu.sync_copy(
                acc_sh.at[pl.ds(sid * FR, FR)],
                out_hbm.at[cid].at[pl.ds(b * NH + sid * FR, FR)])
            plsc.subcore_barrier()

    return deg_kernel


def _make_agg_kernel():
    HCPT = CPT // 2

    @functools.partial(
        pl.kernel,
        mesh=_sc_mesh,
        out_type=jax.ShapeDtypeStruct((NQ, N_PAD, QW), jnp.float32),
        scratch_types=[
            pltpu.VMEM((CPT, CHUNK), jnp.int32),
            pltpu.VMEM((CPT, CHUNK), jnp.int32),
            pltpu.VMEM((CPT, CHUNK), jnp.int32),
            pltpu.VMEM((CHUNK, QW), jnp.float32),
            pltpu.VMEM((CHUNK, QW), jnp.float32),
            pltpu.VMEM_SHARED((ACC_ROWS, QW), jnp.float32),
            pltpu.SemaphoreType.DMA,
            pltpu.SemaphoreType.DMA,
        ],
    )
    def agg_kernel(tab_hbm, src_hbm, dst_hbm, zeros_hbm, out_hbm,
                   src_v, dst_v, dloc_v, rows0, rows1, acc_sh, sem0, sem1):
        cid = lax.axis_index("c")
        sid = lax.axis_index("s")
        pltpu.sync_copy(src_hbm.at[pl.ds(sid * CPT, CPT)], src_v)
        pltpu.sync_copy(dst_hbm.at[pl.ds(sid * CPT, CPT)], dst_v)
        for b in range(2):
            @pl.loop(0, CPT)
            def _(r):
                @pl.loop(0, CHUNK // 16)
                def _(c):
                    v = dst_v[r, pl.ds(c * 16, 16)]
                    loc = v - b * NH
                    ok = (loc >= 0) & (loc < NH)
                    scr = NH + jnp.bitwise_and(v, SCR - 1)
                    dloc_v[r, pl.ds(c * 16, 16)] = jnp.where(ok, loc, scr)

            for q in range(NQ // 2):
                qg = cid * (NQ // 2) + q
                tab_q = tab_hbm.at[qg]
                pltpu.sync_copy(zeros_hbm, acc_sh.at[pl.ds(sid * ZR, ZR)])
                plsc.subcore_barrier()
                pltpu.async_copy(tab_q.at[src_v.at[0]], rows0, sem0)

                @pl.loop(0, HCPT)
                def _(j):
                    i0 = 2 * j
                    pltpu.async_copy(tab_q.at[src_v.at[i0 + 1]], rows1, sem1)
                    pltpu.make_async_copy(
                        tab_q.at[src_v.at[i0]], rows0, sem0).wait()
                    pltpu.sync_copy(rows0, acc_sh.at[dloc_v.at[i0]], add=True)

                    @pl.when(j < HCPT - 1)
                    def _():
                        pltpu.async_copy(
                            tab_q.at[src_v.at[i0 + 2]], rows0, sem0)

                    pltpu.make_async_copy(
                        tab_q.at[src_v.at[i0 + 1]], rows1, sem1).wait()
                    pltpu.sync_copy(rows1, acc_sh.at[dloc_v.at[i0 + 1]],
                                    add=True)

                plsc.subcore_barrier()
                pltpu.sync_copy(
                    acc_sh.at[pl.ds(sid * FR, FR)],
                    out_hbm.at[qg].at[pl.ds(b * NH + sid * FR, FR)])
                plsc.subcore_barrier()

    return agg_kernel


def _prep_kernel(feat_ref, dego_ref, degi_ref, ft_ref, a_ref, b_ref):
    a = lax.rsqrt(jnp.maximum(dego_ref[0][:, :1], 1.0))
    b = lax.rsqrt(jnp.maximum(degi_ref[0][:, :1], 1.0))
    a_ref[...] = a
    b_ref[...] = b
    ft = feat_ref[...] * a
    for q in range(NQ):
        if q < IN_DIM // QW:
            ft_ref[q] = ft[:, q * QW:(q + 1) * QW]
        else:
            ft_ref[q] = jnp.zeros((ft.shape[0], QW), jnp.float32)


def _prep(features_p, deg16):
    return pl.pallas_call(
        _prep_kernel,
        grid=(N_PAD // ROW_BLK,),
        in_specs=[
            pl.BlockSpec((ROW_BLK, IN_DIM), lambda i: (i, 0)),
            pl.BlockSpec((1, ROW_BLK, QW), lambda i: (0, i, 0)),
            pl.BlockSpec((1, ROW_BLK, QW), lambda i: (1, i, 0)),
        ],
        out_specs=[
            pl.BlockSpec((NQ, ROW_BLK, QW), lambda i: (0, i, 0)),
            pl.BlockSpec((ROW_BLK, 1), lambda i: (i, 0)),
            pl.BlockSpec((ROW_BLK, 1), lambda i: (i, 0)),
        ],
        out_shape=[
            jax.ShapeDtypeStruct((NQ, N_PAD, QW), jnp.float32),
            jax.ShapeDtypeStruct((N_PAD, 1), jnp.float32),
            jax.ShapeDtypeStruct((N_PAD, 1), jnp.float32),
        ],
    )(features_p, deg16, deg16)


def _layer_kernel(emit_ht, s_ref, a_ref, b_ref, w_ref, bias_ref, *out):
    x = jnp.concatenate([s_ref[q] for q in range(NQ)], axis=-1)
    x = x * b_ref[...]
    h = jnp.maximum(
        jnp.dot(x, w_ref[...], preferred_element_type=jnp.float32)
        + bias_ref[...], 0.0)
    out[0][...] = h
    if emit_ht:
        ht = h * a_ref[...]
        for q in range(NQ):
            out[1][q] = ht[:, q * QW:(q + 1) * QW]


def _layer(s, a_col, b_col, W, bias, emit_ht):
    out_shape = [jax.ShapeDtypeStruct((N_PAD, HID), jnp.float32)]
    out_specs = [pl.BlockSpec((ROW_BLK, HID), lambda i: (i, 0))]
    if emit_ht:
        out_shape.append(
            jax.ShapeDtypeStruct((NQ, N_PAD, QW), jnp.float32))
        out_specs.append(
            pl.BlockSpec((NQ, ROW_BLK, QW), lambda i: (0, i, 0)))
    return pl.pallas_call(
        functools.partial(_layer_kernel, emit_ht),
        grid=(N_PAD // ROW_BLK,),
        in_specs=[
            pl.BlockSpec((NQ, ROW_BLK, QW), lambda i: (0, i, 0)),
            pl.BlockSpec((ROW_BLK, 1), lambda i: (i, 0)),
            pl.BlockSpec((ROW_BLK, 1), lambda i: (i, 0)),
            pl.BlockSpec((HID, HID), lambda i: (0, 0)),
            pl.BlockSpec((1, HID), lambda i: (0, 0)),
        ],
        out_specs=out_specs,
        out_shape=out_shape,
    )(s, a_col, b_col, W, bias.reshape(1, HID))


def _pool_kernel(s_ref, a_ref, b_ref, w_ref, bias_ref,
                 h1_ref, h2_ref, h3_ref, wa_ref, ba_ref, wo_ref,
                 bo_ref, out_ref):
    x = jnp.concatenate([s_ref[q] for q in range(NQ)], axis=-1) * b_ref[...]
    h4 = jnp.maximum(
        jnp.dot(x, w_ref[...], preferred_element_type=jnp.float32)
        + bias_ref[...], 0.0)
    hs = [h1_ref[...], h2_ref[...], h3_ref[...], h4]
    wa = wa_ref[...]
    scores = [jnp.sum(h * wa, axis=1, keepdims=True) + ba_ref[0, 0] for h in hs]
    m = functools.reduce(jnp.maximum, scores)
    es = [jnp.exp(s - m) for s in scores]
    denom = functools.reduce(jnp.add, es)
    pooled = functools.reduce(
        jnp.add, [h * (e / denom) for h, e in zip(hs, es)])
    out_ref[...] = jnp.dot(pooled, wo_ref[...],
                           preferred_element_type=jnp.float32) + bo_ref[...]


def _pool_out(s3, a_col, b_col, W3, b3, h1, h2, h3, Wa, ba, Wo, bo):
    return pl.pallas_call(
        _pool_kernel,
        grid=(N_PAD // ROW_BLK,),
        in_specs=[
            pl.BlockSpec((NQ, ROW_BLK, QW), lambda i: (0, i, 0)),
            pl.BlockSpec((ROW_BLK, 1), lambda i: (i, 0)),
            pl.BlockSpec((ROW_BLK, 1), lambda i: (i, 0)),
            pl.BlockSpec((HID, HID), lambda i: (0, 0)),
            pl.BlockSpec((1, HID), lambda i: (0, 0)),
        ] + [pl.BlockSpec((ROW_BLK, HID), lambda i: (i, 0))] * 3 + [
            pl.BlockSpec((1, HID), lambda i: (0, 0)),
            pl.BlockSpec((1, 1), lambda i: (0, 0), memory_space=pltpu.SMEM),
            pl.BlockSpec((HID, N_CLASSES), lambda i: (0, 0)),
            pl.BlockSpec((1, N_CLASSES), lambda i: (0, 0)),
        ],
        out_specs=pl.BlockSpec((ROW_BLK, N_CLASSES), lambda i: (i, 0)),
        out_shape=jax.ShapeDtypeStruct((N_PAD, N_CLASSES), jnp.float32),
    )(s3, a_col, b_col, W3, b3.reshape(1, HID), h1, h2, h3,
      Wa.reshape(1, HID), ba.reshape(1, 1), Wo, bo.reshape(1, N_CLASSES))


def kernel(features, edge_index, W0, b0, W1, b1, W2, b2, W3, b3, Wa, ba, Wo, bo):
    src = edge_index[0].astype(jnp.int32)
    dst = edge_index[1].astype(jnp.int32)
    n_fill = E_CAP - N_EDGES
    ar = jnp.arange(n_fill, dtype=jnp.int32)
    src2d = jnp.concatenate(
        [src, N_NODES + (ar % (N_PAD - N_NODES))]).reshape(N_CHUNKS, CHUNK)
    dst2d = jnp.concatenate(
        [dst, 2 * N_PAD + (ar % SCR)]).reshape(N_CHUNKS, CHUNK)
    idx2 = jnp.stack([src2d, dst2d])

    zeros_acc = jnp.zeros((ZR, QW), jnp.float32)
    ones128 = jnp.ones((CHUNK, QW), jnp.float32)
    feats_p = jnp.pad(features, ((0, N_PAD - N_NODES), (0, 0)))
    W0p = jnp.pad(W0, ((0, HID - IN_DIM), (0, 0)))

    deg16 = _make_deg_kernel()(idx2, ones128, zeros_acc)

    ft, a_col, b_col = _prep(feats_p, deg16)

    agg = _make_agg_kernel()

    s0 = agg(ft, src2d, dst2d, zeros_acc)
    h1, ht1 = _layer(s0, a_col, b_col, W0p, b0, True)
    s1 = agg(ht1, src2d, dst2d, zeros_acc)
    h2, ht2 = _layer(s1, a_col, b_col, W1, b1, True)
    s2 = agg(ht2, src2d, dst2d, zeros_acc)
    h3, ht3 = _layer(s2, a_col, b_col, W2, b2, True)
    s3 = agg(ht3, src2d, dst2d, zeros_acc)
    out = _pool_out(s3, a_col, b_col, W3, b3, h1, h2, h3, Wa, ba, Wo, bo)
    return out[:N_NODES]


# speedup vs baseline: 1.5087x; 1.1210x over previous
import functools

import jax
import jax.numpy as jnp
from jax import lax
from jax.experimental import pallas as pl
from jax.experimental.pallas import tpu as pltpu
from jax.experimental.pallas import tpu_sc as plsc

N_NODES = 10000
N_PAD = 10240
ROW_BLK = 512
HID = 512
IN_DIM = 256
N_CLASSES = 128
QW = 128
NQ = HID // QW

N_EDGES = 160000
CHUNK = 128
N_CHUNKS = 1280
E_CAP = N_CHUNKS * CHUNK
NS = 16
CPT = N_CHUNKS // NS

NH = N_PAD // 2
SCR = 1024
ACC_ROWS = NH + SCR
ZR = ACC_ROWS // NS
FR = NH // NS

_sc_mesh = plsc.VectorSubcoreMesh(core_axis_name="c", subcore_axis_name="s")


def _make_deg_kernel():
    @functools.partial(
        pl.kernel,
        mesh=_sc_mesh,
        out_type=jax.ShapeDtypeStruct((2, N_PAD, QW), jnp.float32),
        scratch_types=[
            pltpu.VMEM((CPT, CHUNK), jnp.int32),
            pltpu.VMEM((CPT, CHUNK), jnp.int32),
            pltpu.VMEM((CHUNK, QW), jnp.float32),
            pltpu.VMEM_SHARED((ACC_ROWS, QW), jnp.float32),
        ],
    )
    def deg_kernel(idx_hbm, ones_hbm, zeros_hbm, out_hbm,
                   idx_v, iloc_v, ones_v, acc_sh):
        cid = lax.axis_index("c")
        sid = lax.axis_index("s")
        pltpu.sync_copy(ones_hbm, ones_v)
        pltpu.sync_copy(idx_hbm.at[cid].at[pl.ds(sid * CPT, CPT)], idx_v)
        for b in range(2):
            @pl.loop(0, CPT)
            def _(r):
                @pl.loop(0, CHUNK // 16)
                def _(c):
                    v = idx_v[r, pl.ds(c * 16, 16)]
                    loc = v - b * NH
                    ok = (loc >= 0) & (loc < NH)
                    scr = NH + jnp.bitwise_and(v, SCR - 1)
                    iloc_v[r, pl.ds(c * 16, 16)] = jnp.where(ok, loc, scr)

            pltpu.sync_copy(zeros_hbm, acc_sh.at[pl.ds(sid * ZR, ZR)])
            plsc.subcore_barrier()

            @pl.loop(0, CPT)
            def _(i):
                pltpu.sync_copy(ones_v, acc_sh.at[iloc_v.at[i]], add=True)

            plsc.subcore_barrier()
            pltpu.sync_copy(
                acc_sh.at[pl.ds(sid * FR, FR)],
                out_hbm.at[cid].at[pl.ds(b * NH + sid * FR, FR)])
            plsc.subcore_barrier()

    return deg_kernel


def _make_agg_kernel(nq):
    HCPT = CPT // 2

    @functools.partial(
        pl.kernel,
        mesh=_sc_mesh,
        out_type=jax.ShapeDtypeStruct((nq, N_PAD, QW), jnp.float32),
        scratch_types=[
            pltpu.VMEM((CPT, CHUNK), jnp.int32),
            pltpu.VMEM((CPT, CHUNK), jnp.int32),
            pltpu.VMEM((CPT, CHUNK), jnp.int32),
            pltpu.VMEM((CHUNK, QW), jnp.float32),
            pltpu.VMEM((CHUNK, QW), jnp.float32),
            pltpu.VMEM_SHARED((ACC_ROWS, QW), jnp.float32),
            pltpu.SemaphoreType.DMA,
            pltpu.SemaphoreType.DMA,
        ],
    )
    def agg_kernel(tab_hbm, src_hbm, dst_hbm, zeros_hbm, out_hbm,
                   src_v, dst_v, dloc_v, rows0, rows1, acc_sh, sem0, sem1):
        cid = lax.axis_index("c")
        sid = lax.axis_index("s")
        pltpu.sync_copy(src_hbm.at[pl.ds(sid * CPT, CPT)], src_v)
        pltpu.sync_copy(dst_hbm.at[pl.ds(sid * CPT, CPT)], dst_v)
        for b in range(2):
            @pl.loop(0, CPT)
            def _(r):
                @pl.loop(0, CHUNK // 16)
                def _(c):
                    v = dst_v[r, pl.ds(c * 16, 16)]
                    loc = v - b * NH
                    ok = (loc >= 0) & (loc < NH)
                    scr = NH + jnp.bitwise_and(v, SCR - 1)
                    dloc_v[r, pl.ds(c * 16, 16)] = jnp.where(ok, loc, scr)

            for q in range(nq // 2):
                qg = cid * (nq // 2) + q
                tab_q = tab_hbm.at[qg]
                pltpu.sync_copy(zeros_hbm, acc_sh.at[pl.ds(sid * ZR, ZR)])
                plsc.subcore_barrier()
                pltpu.async_copy(tab_q.at[src_v.at[0]], rows0, sem0)

                @pl.loop(0, HCPT)
                def _(j):
                    i0 = 2 * j
                    pltpu.async_copy(tab_q.at[src_v.at[i0 + 1]], rows1, sem1)
                    pltpu.make_async_copy(
                        tab_q.at[src_v.at[i0]], rows0, sem0).wait()
                    pltpu.sync_copy(rows0, acc_sh.at[dloc_v.at[i0]], add=True)

                    @pl.when(j < HCPT - 1)
                    def _():
                        pltpu.async_copy(
                            tab_q.at[src_v.at[i0 + 2]], rows0, sem0)

                    pltpu.make_async_copy(
                        tab_q.at[src_v.at[i0 + 1]], rows1, sem1).wait()
                    pltpu.sync_copy(rows1, acc_sh.at[dloc_v.at[i0 + 1]],
                                    add=True)

                plsc.subcore_barrier()
                pltpu.sync_copy(
                    acc_sh.at[pl.ds(sid * FR, FR)],
                    out_hbm.at[qg].at[pl.ds(b * NH + sid * FR, FR)])
                plsc.subcore_barrier()

    return agg_kernel


def _prep_kernel(feat_ref, dego_ref, degi_ref, ft_ref, a_ref, b_ref):
    a = lax.rsqrt(jnp.maximum(dego_ref[0][:, :1], 1.0))
    b = lax.rsqrt(jnp.maximum(degi_ref[0][:, :1], 1.0))
    a_ref[...] = a
    b_ref[...] = b
    ft = feat_ref[...] * a
    for q in range(IN_DIM // QW):
        ft_ref[q] = ft[:, q * QW:(q + 1) * QW]


def _prep(features_p, deg16):
    return pl.pallas_call(
        _prep_kernel,
        grid=(N_PAD // ROW_BLK,),
        in_specs=[
            pl.BlockSpec((ROW_BLK, IN_DIM), lambda i: (i, 0)),
            pl.BlockSpec((1, ROW_BLK, QW), lambda i: (0, i, 0)),
            pl.BlockSpec((1, ROW_BLK, QW), lambda i: (1, i, 0)),
        ],
        out_specs=[
            pl.BlockSpec((IN_DIM // QW, ROW_BLK, QW), lambda i: (0, i, 0)),
            pl.BlockSpec((ROW_BLK, 1), lambda i: (i, 0)),
            pl.BlockSpec((ROW_BLK, 1), lambda i: (i, 0)),
        ],
        out_shape=[
            jax.ShapeDtypeStruct((IN_DIM // QW, N_PAD, QW), jnp.float32),
            jax.ShapeDtypeStruct((N_PAD, 1), jnp.float32),
            jax.ShapeDtypeStruct((N_PAD, 1), jnp.float32),
        ],
    )(features_p, deg16, deg16)


def _layer_kernel(nq_in, emit_ht, s_ref, a_ref, b_ref, w_ref, bias_ref, *out):
    x = jnp.concatenate([s_ref[q] for q in range(nq_in)], axis=-1)
    x = x * b_ref[...]
    h = jnp.maximum(
        jnp.dot(x, w_ref[...], preferred_element_type=jnp.float32)
        + bias_ref[...], 0.0)
    out[0][...] = h
    if emit_ht:
        ht = h * a_ref[...]
        for q in range(NQ):
            out[1][q] = ht[:, q * QW:(q + 1) * QW]


def _layer(s, a_col, b_col, W, bias, emit_ht):
    out_shape = [jax.ShapeDtypeStruct((N_PAD, HID), jnp.float32)]
    out_specs = [pl.BlockSpec((ROW_BLK, HID), lambda i: (i, 0))]
    if emit_ht:
        out_shape.append(
            jax.ShapeDtypeStruct((NQ, N_PAD, QW), jnp.float32))
        out_specs.append(
            pl.BlockSpec((NQ, ROW_BLK, QW), lambda i: (0, i, 0)))
    nq_in = s.shape[0]
    return pl.pallas_call(
        functools.partial(_layer_kernel, nq_in, emit_ht),
        grid=(N_PAD // ROW_BLK,),
        in_specs=[
            pl.BlockSpec((nq_in, ROW_BLK, QW), lambda i: (0, i, 0)),
            pl.BlockSpec((ROW_BLK, 1), lambda i: (i, 0)),
            pl.BlockSpec((ROW_BLK, 1), lambda i: (i, 0)),
            pl.BlockSpec((nq_in * QW, HID), lambda i: (0, 0)),
            pl.BlockSpec((1, HID), lambda i: (0, 0)),
        ],
        out_specs=out_specs,
        out_shape=out_shape,
    )(s, a_col, b_col, W, bias.reshape(1, HID))


def _pool_kernel(s_ref, a_ref, b_ref, w_ref, bias_ref,
                 h1_ref, h2_ref, h3_ref, wa_ref, ba_ref, wo_ref,
                 bo_ref, out_ref):
    x = jnp.concatenate([s_ref[q] for q in range(NQ)], axis=-1) * b_ref[...]
    h4 = jnp.maximum(
        jnp.dot(x, w_ref[...], preferred_element_type=jnp.float32)
        + bias_ref[...], 0.0)
    hs = [h1_ref[...], h2_ref[...], h3_ref[...], h4]
    wa = wa_ref[...]
    scores = [jnp.sum(h * wa, axis=1, keepdims=True) + ba_ref[0, 0] for h in hs]
    m = functools.reduce(jnp.maximum, scores)
    es = [jnp.exp(s - m) for s in scores]
    denom = functools.reduce(jnp.add, es)
    pooled = functools.reduce(
        jnp.add, [h * (e / denom) for h, e in zip(hs, es)])
    out_ref[...] = jnp.dot(pooled, wo_ref[...],
                           preferred_element_type=jnp.float32) + bo_ref[...]


def _pool_out(s3, a_col, b_col, W3, b3, h1, h2, h3, Wa, ba, Wo, bo):
    return pl.pallas_call(
        _pool_kernel,
        grid=(N_PAD // ROW_BLK,),
        in_specs=[
            pl.BlockSpec((NQ, ROW_BLK, QW), lambda i: (0, i, 0)),
            pl.BlockSpec((ROW_BLK, 1), lambda i: (i, 0)),
            pl.BlockSpec((ROW_BLK, 1), lambda i: (i, 0)),
            pl.BlockSpec((HID, HID), lambda i: (0, 0)),
            pl.BlockSpec((1, HID), lambda i: (0, 0)),
        ] + [pl.BlockSpec((ROW_BLK, HID), lambda i: (i, 0))] * 3 + [
            pl.BlockSpec((1, HID), lambda i: (0, 0)),
            pl.BlockSpec((1, 1), lambda i: (0, 0), memory_space=pltpu.SMEM),
            pl.BlockSpec((HID, N_CLASSES), lambda i: (0, 0)),
            pl.BlockSpec((1, N_CLASSES), lambda i: (0, 0)),
        ],
        out_specs=pl.BlockSpec((ROW_BLK, N_CLASSES), lambda i: (i, 0)),
        out_shape=jax.ShapeDtypeStruct((N_PAD, N_CLASSES), jnp.float32),
    )(s3, a_col, b_col, W3, b3.reshape(1, HID), h1, h2, h3,
      Wa.reshape(1, HID), ba.reshape(1, 1), Wo, bo.reshape(1, N_CLASSES))


def kernel(features, edge_index, W0, b0, W1, b1, W2, b2, W3, b3, Wa, ba, Wo, bo):
    src = edge_index[0].astype(jnp.int32)
    dst = edge_index[1].astype(jnp.int32)
    n_fill = E_CAP - N_EDGES
    ar = jnp.arange(n_fill, dtype=jnp.int32)
    src2d = jnp.concatenate(
        [src, N_NODES + (ar % (N_PAD - N_NODES))]).reshape(N_CHUNKS, CHUNK)
    dst2d = jnp.concatenate(
        [dst, 2 * N_PAD + (ar % SCR)]).reshape(N_CHUNKS, CHUNK)
    idx2 = jnp.stack([src2d, dst2d])

    zeros_acc = jnp.zeros((ZR, QW), jnp.float32)
    ones128 = jnp.ones((CHUNK, QW), jnp.float32)
    feats_p = jnp.pad(features, ((0, N_PAD - N_NODES), (0, 0)))

    deg16 = _make_deg_kernel()(idx2, ones128, zeros_acc)

    ft, a_col, b_col = _prep(feats_p, deg16)

    agg0 = _make_agg_kernel(IN_DIM // QW)
    agg = _make_agg_kernel(NQ)

    s0 = agg0(ft, src2d, dst2d, zeros_acc)
    h1, ht1 = _layer(s0, a_col, b_col, W0, b0, True)
    s1 = agg(ht1, src2d, dst2d, zeros_acc)
    h2, ht2 = _layer(s1, a_col, b_col, W1, b1, True)
    s2 = agg(ht2, src2d, dst2d, zeros_acc)
    h3, ht3 = _layer(s2, a_col, b_col, W2, b2, True)
    s3 = agg(ht3, src2d, dst2d, zeros_acc)
    out = _pool_out(s3, a_col, b_col, W3, b3, h1, h2, h3, Wa, ba, Wo, bo)
    return out[:N_NODES]


# speedup vs baseline: 1.5092x; 1.0003x over previous
import functools

import jax
import jax.numpy as jnp
from jax import lax
from jax.experimental import pallas as pl
from jax.experimental.pallas import tpu as pltpu
from jax.experimental.pallas import tpu_sc as plsc

N_NODES = 10000
N_PAD = 10240
ROW_BLK = 512
HID = 512
IN_DIM = 256
N_CLASSES = 128
QW = 128
NQ = HID // QW

N_EDGES = 160000
CHUNK = 128
N_CHUNKS = 1280
E_CAP = N_CHUNKS * CHUNK
NS = 16
CPT = N_CHUNKS // NS

NH = N_PAD // 2
SCR = 1024
ACC_ROWS = NH + SCR
ZR = ACC_ROWS // NS
FR = NH // NS

_sc_mesh = plsc.VectorSubcoreMesh(core_axis_name="c", subcore_axis_name="s")


def _make_deg_kernel():
    @functools.partial(
        pl.kernel,
        mesh=_sc_mesh,
        out_type=jax.ShapeDtypeStruct((2, N_PAD, QW), jnp.float32),
        scratch_types=[
            pltpu.VMEM((CPT, CHUNK), jnp.int32),
            pltpu.VMEM((CPT, CHUNK), jnp.int32),
            pltpu.VMEM((CHUNK, QW), jnp.float32),
            pltpu.VMEM_SHARED((ACC_ROWS, QW), jnp.float32),
        ],
    )
    def deg_kernel(idx_hbm, ones_hbm, zeros_hbm, out_hbm,
                   idx_v, iloc_v, ones_v, acc_sh):
        cid = lax.axis_index("c")
        sid = lax.axis_index("s")
        pltpu.sync_copy(ones_hbm, ones_v)
        pltpu.sync_copy(idx_hbm.at[cid].at[pl.ds(sid * CPT, CPT)], idx_v)
        for b in range(2):
            @pl.loop(0, CPT)
            def _(r):
                @pl.loop(0, CHUNK // 16)
                def _(c):
                    v = idx_v[r, pl.ds(c * 16, 16)]
                    loc = v - b * NH
                    ok = (loc >= 0) & (loc < NH)
                    scr = NH + jnp.bitwise_and(v, SCR - 1)
                    iloc_v[r, pl.ds(c * 16, 16)] = jnp.where(ok, loc, scr)

            pltpu.sync_copy(zeros_hbm, acc_sh.at[pl.ds(sid * ZR, ZR)])
            plsc.subcore_barrier()

            @pl.loop(0, CPT)
            def _(i):
                pltpu.sync_copy(ones_v, acc_sh.at[iloc_v.at[i]], add=True)

            plsc.subcore_barrier()
            pltpu.sync_copy(
                acc_sh.at[pl.ds(sid * FR, FR)],
                out_hbm.at[cid].at[pl.ds(b * NH + sid * FR, FR)])
            plsc.subcore_barrier()

    return deg_kernel


def _make_agg_kernel(nq):
    HCPT = CPT // 2

    @functools.partial(
        pl.kernel,
        mesh=_sc_mesh,
        out_type=jax.ShapeDtypeStruct((nq, N_PAD, QW), jnp.float32),
        scratch_types=[
            pltpu.VMEM((CPT, CHUNK), jnp.int32),
            pltpu.VMEM((CPT, CHUNK), jnp.int32),
            pltpu.VMEM((CPT, CHUNK), jnp.int32),
            pltpu.VMEM((CHUNK, QW), jnp.float32),
            pltpu.VMEM((CHUNK, QW), jnp.float32),
            pltpu.VMEM_SHARED((ACC_ROWS, QW), jnp.float32),
            pltpu.SemaphoreType.DMA,
            pltpu.SemaphoreType.DMA,
            pltpu.SemaphoreType.DMA,
        ],
    )
    def agg_kernel(tab_hbm, src_hbm, dst_hbm, zeros_hbm, out_hbm,
                   src_v, dst_v, dloc_v, rows0, rows1, acc_sh,
                   sem0, sem1, ssem):
        cid = lax.axis_index("c")
        sid = lax.axis_index("s")
        pltpu.sync_copy(src_hbm.at[pl.ds(sid * CPT, CPT)], src_v)
        pltpu.sync_copy(dst_hbm.at[pl.ds(sid * CPT, CPT)], dst_v)
        for b in range(2):
            @pl.loop(0, CPT)
            def _(r):
                @pl.loop(0, CHUNK // 16)
                def _(c):
                    v = dst_v[r, pl.ds(c * 16, 16)]
                    loc = v - b * NH
                    ok = (loc >= 0) & (loc < NH)
                    scr = NH + jnp.bitwise_and(v, SCR - 1)
                    dloc_v[r, pl.ds(c * 16, 16)] = jnp.where(ok, loc, scr)

            for q in range(nq // 2):
                qg = cid * (nq // 2) + q
                tab_q = tab_hbm.at[qg]
                pltpu.sync_copy(zeros_hbm, acc_sh.at[pl.ds(sid * ZR, ZR)])
                plsc.subcore_barrier()
                pltpu.async_copy(tab_q.at[src_v.at[0]], rows0, sem0)

                @pl.loop(0, HCPT)
                def _(j):
                    i0 = 2 * j

                    @pl.when(j >= 1)
                    def _():
                        pltpu.make_async_copy(
                            tab_q.at[pl.ds(0, CHUNK)], rows1, ssem).wait()

                    pltpu.async_copy(tab_q.at[src_v.at[i0 + 1]], rows1, sem1)
                    pltpu.make_async_copy(
                        tab_q.at[src_v.at[i0]], rows0, sem0).wait()
                    pltpu.sync_copy(rows0, acc_sh.at[dloc_v.at[i0]], add=True)

                    @pl.when(j < HCPT - 1)
                    def _():
                        pltpu.async_copy(
                            tab_q.at[src_v.at[i0 + 2]], rows0, sem0)

                    pltpu.make_async_copy(
                        tab_q.at[src_v.at[i0 + 1]], rows1, sem1).wait()
                    pltpu.async_copy(rows1, acc_sh.at[dloc_v.at[i0 + 1]],
                                     ssem, add=True)

                pltpu.make_async_copy(
                    tab_q.at[pl.ds(0, CHUNK)], rows1, ssem).wait()
                plsc.subcore_barrier()
                pltpu.sync_copy(
                    acc_sh.at[pl.ds(sid * FR, FR)],
                    out_hbm.at[qg].at[pl.ds(b * NH + sid * FR, FR)])
                plsc.subcore_barrier()

    return agg_kernel


def _prep_kernel(feat_ref, dego_ref, degi_ref, ft_ref, a_ref, b_ref):
    a = lax.rsqrt(jnp.maximum(dego_ref[0][:, :1], 1.0))
    b = lax.rsqrt(jnp.maximum(degi_ref[0][:, :1], 1.0))
    a_ref[...] = a
    b_ref[...] = b
    ft = feat_ref[...] * a
    for q in range(IN_DIM // QW):
        ft_ref[q] = ft[:, q * QW:(q + 1) * QW]


def _prep(features_p, deg16):
    return pl.pallas_call(
        _prep_kernel,
        grid=(N_PAD // ROW_BLK,),
        in_specs=[
            pl.BlockSpec((ROW_BLK, IN_DIM), lambda i: (i, 0)),
            pl.BlockSpec((1, ROW_BLK, QW), lambda i: (0, i, 0)),
            pl.BlockSpec((1, ROW_BLK, QW), lambda i: (1, i, 0)),
        ],
        out_specs=[
            pl.BlockSpec((IN_DIM // QW, ROW_BLK, QW), lambda i: (0, i, 0)),
            pl.BlockSpec((ROW_BLK, 1), lambda i: (i, 0)),
            pl.BlockSpec((ROW_BLK, 1), lambda i: (i, 0)),
        ],
        out_shape=[
            jax.ShapeDtypeStruct((IN_DIM // QW, N_PAD, QW), jnp.float32),
            jax.ShapeDtypeStruct((N_PAD, 1), jnp.float32),
            jax.ShapeDtypeStruct((N_PAD, 1), jnp.float32),
        ],
    )(features_p, deg16, deg16)


def _layer_kernel(nq_in, emit_ht, s_ref, a_ref, b_ref, w_ref, bias_ref, *out):
    x = jnp.concatenate([s_ref[q] for q in range(nq_in)], axis=-1)
    x = x * b_ref[...]
    h = jnp.maximum(
        jnp.dot(x, w_ref[...], preferred_element_type=jnp.float32)
        + bias_ref[...], 0.0)
    out[0][...] = h
    if emit_ht:
        ht = h * a_ref[...]
        for q in range(NQ):
            out[1][q] = ht[:, q * QW:(q + 1) * QW]


def _layer(s, a_col, b_col, W, bias, emit_ht):
    out_shape = [jax.ShapeDtypeStruct((N_PAD, HID), jnp.float32)]
    out_specs = [pl.BlockSpec((ROW_BLK, HID), lambda i: (i, 0))]
    if emit_ht:
        out_shape.append(
            jax.ShapeDtypeStruct((NQ, N_PAD, QW), jnp.float32))
        out_specs.append(
            pl.BlockSpec((NQ, ROW_BLK, QW), lambda i: (0, i, 0)))
    nq_in = s.shape[0]
    return pl.pallas_call(
        functools.partial(_layer_kernel, nq_in, emit_ht),
        grid=(N_PAD // ROW_BLK,),
        in_specs=[
            pl.BlockSpec((nq_in, ROW_BLK, QW), lambda i: (0, i, 0)),
            pl.BlockSpec((ROW_BLK, 1), lambda i: (i, 0)),
            pl.BlockSpec((ROW_BLK, 1), lambda i: (i, 0)),
            pl.BlockSpec((nq_in * QW, HID), lambda i: (0, 0)),
            pl.BlockSpec((1, HID), lambda i: (0, 0)),
        ],
        out_specs=out_specs,
        out_shape=out_shape,
    )(s, a_col, b_col, W, bias.reshape(1, HID))


def _pool_kernel(s_ref, a_ref, b_ref, w_ref, bias_ref,
                 h1_ref, h2_ref, h3_ref, wa_ref, ba_ref, wo_ref,
                 bo_ref, out_ref):
    x = jnp.concatenate([s_ref[q] for q in range(NQ)], axis=-1) * b_ref[...]
    h4 = jnp.maximum(
        jnp.dot(x, w_ref[...], preferred_element_type=jnp.float32)
        + bias_ref[...], 0.0)
    hs = [h1_ref[...], h2_ref[...], h3_ref[...], h4]
    wa = wa_ref[...]
    scores = [jnp.sum(h * wa, axis=1, keepdims=True) + ba_ref[0, 0] for h in hs]
    m = functools.reduce(jnp.maximum, scores)
    es = [jnp.exp(s - m) for s in scores]
    denom = functools.reduce(jnp.add, es)
    pooled = functools.reduce(
        jnp.add, [h * (e / denom) for h, e in zip(hs, es)])
    out_ref[...] = jnp.dot(pooled, wo_ref[...],
                           preferred_element_type=jnp.float32) + bo_ref[...]


def _pool_out(s3, a_col, b_col, W3, b3, h1, h2, h3, Wa, ba, Wo, bo):
    return pl.pallas_call(
        _pool_kernel,
        grid=(N_PAD // ROW_BLK,),
        in_specs=[
            pl.BlockSpec((NQ, ROW_BLK, QW), lambda i: (0, i, 0)),
            pl.BlockSpec((ROW_BLK, 1), lambda i: (i, 0)),
            pl.BlockSpec((ROW_BLK, 1), lambda i: (i, 0)),
            pl.BlockSpec((HID, HID), lambda i: (0, 0)),
            pl.BlockSpec((1, HID), lambda i: (0, 0)),
        ] + [pl.BlockSpec((ROW_BLK, HID), lambda i: (i, 0))] * 3 + [
            pl.BlockSpec((1, HID), lambda i: (0, 0)),
            pl.BlockSpec((1, 1), lambda i: (0, 0), memory_space=pltpu.SMEM),
            pl.BlockSpec((HID, N_CLASSES), lambda i: (0, 0)),
            pl.BlockSpec((1, N_CLASSES), lambda i: (0, 0)),
        ],
        out_specs=pl.BlockSpec((ROW_BLK, N_CLASSES), lambda i: (i, 0)),
        out_shape=jax.ShapeDtypeStruct((N_PAD, N_CLASSES), jnp.float32),
    )(s3, a_col, b_col, W3, b3.reshape(1, HID), h1, h2, h3,
      Wa.reshape(1, HID), ba.reshape(1, 1), Wo, bo.reshape(1, N_CLASSES))


def kernel(features, edge_index, W0, b0, W1, b1, W2, b2, W3, b3, Wa, ba, Wo, bo):
    src = edge_index[0].astype(jnp.int32)
    dst = edge_index[1].astype(jnp.int32)
    n_fill = E_CAP - N_EDGES
    ar = jnp.arange(n_fill, dtype=jnp.int32)
    src2d = jnp.concatenate(
        [src, N_NODES + (ar % (N_PAD - N_NODES))]).reshape(N_CHUNKS, CHUNK)
    dst2d = jnp.concatenate(
        [dst, 2 * N_PAD + (ar % SCR)]).reshape(N_CHUNKS, CHUNK)
    idx2 = jnp.stack([src2d, dst2d])

    zeros_acc = jnp.zeros((ZR, QW), jnp.float32)
    ones128 = jnp.ones((CHUNK, QW), jnp.float32)
    feats_p = jnp.pad(features, ((0, N_PAD - N_NODES), (0, 0)))

    deg16 = _make_deg_kernel()(idx2, ones128, zeros_acc)

    ft, a_col, b_col = _prep(feats_p, deg16)

    agg0 = _make_agg_kernel(IN_DIM // QW)
    agg = _make_agg_kernel(NQ)

    s0 = agg0(ft, src2d, dst2d, zeros_acc)
    h1, ht1 = _layer(s0, a_col, b_col, W0, b0, True)
    s1 = agg(ht1, src2d, dst2d, zeros_acc)
    h2, ht2 = _layer(s1, a_col, b_col, W1, b1, True)
    s2 = agg(ht2, src2d, dst2d, zeros_acc)
    h3, ht3 = _layer(s2, a_col, b_col, W2, b2, True)
    s3 = agg(ht3, src2d, dst2d, zeros_acc)
    out = _pool_out(s3, a_col, b_col, W3, b3, h1, h2, h3, Wa, ba, Wo, bo)
    return out[:N_NODES]
